# Initial kernel scaffold; baseline (speedup 1.0000x reference)
import jax
import jax.numpy as jnp
from jax.experimental import pallas as pl


def kernel(h, edge_index, edge_attr, We1, be1, We2, be2, Wh1, bh1, Wh2, bh2):
    raise NotImplementedError("write your pallas kernel here")


# speedup vs baseline: 2.1407x; 2.1407x over previous
import functools

import jax
import jax.numpy as jnp
from jax import lax
from jax.experimental import pallas as pl
from jax.experimental.pallas import tpu as pltpu
from jax.experimental.pallas import tpu_sc as plsc

_NN = 10000
_NE = 320000
_D = 128
_NC = 2
_NS = 16
_EPT = _NE // _NS
_C = 80
_CH = _EPT // _C
_RB = 624
_RTAIL = _NN - _NS * _RB


def _proj_body(h_ref, wa_ref, wb_ref, a_ref, b_ref):
    hb = h_ref[...]
    a_ref[...] = jnp.dot(hb, wa_ref[...], preferred_element_type=jnp.float32)
    b_ref[...] = jnp.dot(hb, wb_ref[...], preferred_element_type=jnp.float32)


def _proj(h, wa, wb):
    blk = 1000
    return pl.pallas_call(
        _proj_body,
        grid=(_NN // blk,),
        in_specs=[
            pl.BlockSpec((blk, _D), lambda i: (i, 0)),
            pl.BlockSpec((_D, _D), lambda i: (0, 0)),
            pl.BlockSpec((_D, _D), lambda i: (0, 0)),
        ],
        out_specs=[
            pl.BlockSpec((blk, _D), lambda i: (i, 0)),
            pl.BlockSpec((blk, _D), lambda i: (i, 0)),
        ],
        out_shape=[
            jax.ShapeDtypeStruct((_NN, _D), jnp.float32),
            jax.ShapeDtypeStruct((_NN, _D), jnp.float32),
        ],
    )(h, wa, wb)


def _ebias_body(attr_ref, we_ref, be_ref, e_ref):
    e_ref[...] = (
        jnp.dot(attr_ref[...], we_ref[...], preferred_element_type=jnp.float32)
        + be_ref[...]
    )


def _ebias(attr, wec, be1):
    blk = 2000
    return pl.pallas_call(
        _ebias_body,
        grid=(_NE // blk,),
        in_specs=[
            pl.BlockSpec((blk, 16), lambda i: (i, 0)),
            pl.BlockSpec((16, _D), lambda i: (0, 0)),
            pl.BlockSpec((1, _D), lambda i: (0, 0)),
        ],
        out_specs=pl.BlockSpec((blk, _D), lambda i: (i, 0)),
        out_shape=jax.ShapeDtypeStruct((_NE, _D), jnp.float32),
    )(attr, wec, be1)


def _sc_body(a_hbm, b_hbm, e_hbm, src_hbm, dst_hbm,
             s2_out, c2_out,
             sidx, didx, abuf, bbuf, ebuf, ones_v, zcnt,
             s_acc, c_acc, sem_a, sem_b, sem_e):
    s = lax.axis_index("s")
    wid = s
    tid = s

    zero16 = jnp.zeros((16,), jnp.float32)
    one16 = jnp.full((16,), 1.0, jnp.float32)

    def _fill_row(i, _):
        for j in range(_D // 16):
            ebuf[i, pl.ds(j * 16, 16)] = zero16
        return _
    lax.fori_loop(0, _C, _fill_row, None)

    def _fill_ones(i, _):
        ones_v[pl.ds(i * 16, 16)] = one16
        return _
    lax.fori_loop(0, _C // 16, _fill_ones, None)

    def _fill_zc(i, _):
        zcnt[pl.ds(i * 16, 16)] = zero16
        return _
    lax.fori_loop(0, _RB // 16, _fill_zc, None)

    base_r = tid * _RB
    pltpu.sync_copy(ebuf, s_acc.at[pl.ds(base_r, _C)])
    pltpu.sync_copy(ebuf, s_acc.at[pl.ds(base_r + _C, _C)])
    pltpu.sync_copy(ebuf, s_acc.at[pl.ds(base_r + 2 * _C, _C)])
    pltpu.sync_copy(ebuf.at[pl.ds(0, _RB - 3 * _C)],
                    s_acc.at[pl.ds(base_r + 3 * _C, _RB - 3 * _C)])
    pltpu.sync_copy(zcnt, c_acc.at[pl.ds(base_r, _RB)])

    @pl.when(tid == _NS - 1)
    def _zero_tail():
        pltpu.sync_copy(ebuf.at[pl.ds(0, _RTAIL)],
                        s_acc.at[pl.ds(_NS * _RB, _RTAIL)])
        pltpu.sync_copy(zcnt.at[pl.ds(0, _RTAIL)],
                        c_acc.at[pl.ds(_NS * _RB, _RTAIL)])
    plsc.subcore_barrier()

    def _chunk(k, _):
        base = pl.multiple_of(wid * _EPT + k * _C, 8)
        pltpu.sync_copy(src_hbm.at[pl.ds(base, _C)], sidx)
        pltpu.sync_copy(dst_hbm.at[pl.ds(base, _C)], didx)
        ca = pltpu.async_copy(a_hbm.at[sidx], abuf, sem_a)
        cb = pltpu.async_copy(b_hbm.at[didx], bbuf, sem_b)
        ce = pltpu.async_copy(e_hbm.at[pl.ds(base, _C)], ebuf, sem_e)
        ca.wait()
        cb.wait()
        ce.wait()

        def _row(i, _):
            for j in range(_D // 16):
                sl = pl.ds(j * 16, 16)
                abuf[i, sl] = jnp.maximum(
                    abuf[i, sl] + bbuf[i, sl] + ebuf[i, sl], 0.0)
            return _
        lax.fori_loop(0, _C, _row, None)

        pltpu.sync_copy(abuf, s_acc.at[didx], add=True)
        pltpu.sync_copy(ones_v, c_acc.at[didx], add=True)
        return _
    lax.fori_loop(0, _CH, _chunk, None)

    plsc.subcore_barrier()

    rows = pl.ds(base_r, _RB)
    pltpu.sync_copy(s_acc.at[rows], s2_out.at[rows])
    pltpu.sync_copy(c_acc.at[rows], zcnt)
    pltpu.sync_copy(zcnt, c2_out.at[pl.ds(base_r, _RB)])

    @pl.when(tid == _NS - 1)
    def _write_tail():
        tail = pl.ds(_NS * _RB, _RTAIL)
        pltpu.sync_copy(s_acc.at[tail], s2_out.at[tail])
        pltpu.sync_copy(c_acc.at[tail], zcnt.at[pl.ds(0, _RTAIL)])
        pltpu.sync_copy(zcnt.at[pl.ds(0, _RTAIL)],
                        c2_out.at[pl.ds(_NS * _RB, _RTAIL)])


_sc_agg = pl.kernel(
    _sc_body,
    out_type=(
        jax.ShapeDtypeStruct((_NN, _D), jnp.float32),
        jax.ShapeDtypeStruct((_NN,), jnp.float32),
    ),
    mesh=plsc.VectorSubcoreMesh(
        core_axis_name="c", subcore_axis_name="s",
        num_cores=1, num_subcores=_NS),
    scratch_types=[
        pltpu.VMEM((_C,), jnp.int32),
        pltpu.VMEM((_C,), jnp.int32),
        pltpu.VMEM((_C, _D), jnp.float32),
        pltpu.VMEM((_C, _D), jnp.float32),
        pltpu.VMEM((_C, _D), jnp.float32),
        pltpu.VMEM((_C,), jnp.float32),
        pltpu.VMEM((_RB,), jnp.float32),
        pltpu.VMEM_SHARED((_NN, _D), jnp.float32),
        pltpu.VMEM_SHARED((_NN,), jnp.float32),
        pltpu.SemaphoreType.DMA,
        pltpu.SemaphoreType.DMA,
        pltpu.SemaphoreType.DMA,
    ],
)


def _node_body(s2_ref, c2_ref, h_ref, we2_ref, be2_ref,
               wh1a_ref, wh1b_ref, bh1_ref, wh2_ref, bh2_ref, o_ref):
    S = s2_ref[...]
    cnt = c2_ref[...]
    m = (jnp.dot(S, we2_ref[...], preferred_element_type=jnp.float32)
         + cnt * be2_ref[...])
    u = jnp.maximum(
        jnp.dot(h_ref[...], wh1a_ref[...], preferred_element_type=jnp.float32)
        + jnp.dot(m, wh1b_ref[...], preferred_element_type=jnp.float32)
        + bh1_ref[...], 0.0)
    o_ref[...] = (jnp.dot(u, wh2_ref[...], preferred_element_type=jnp.float32)
                  + bh2_ref[...])


def _node(s2, c2, h, we2, be2, wh1a, wh1b, bh1, wh2, bh2):
    blk = 1000
    full = lambda i: (0, 0)
    return pl.pallas_call(
        _node_body,
        grid=(_NN // blk,),
        in_specs=[
            pl.BlockSpec((blk, _D), lambda i: (i, 0)),
            pl.BlockSpec((blk, 1), lambda i: (i, 0)),
            pl.BlockSpec((blk, _D), lambda i: (i, 0)),
            pl.BlockSpec((_D, _D), full),
            pl.BlockSpec((1, _D), full),
            pl.BlockSpec((_D, _D), full),
            pl.BlockSpec((_D, _D), full),
            pl.BlockSpec((1, _D), full),
            pl.BlockSpec((_D, _D), full),
            pl.BlockSpec((1, _D), full),
        ],
        out_specs=pl.BlockSpec((blk, _D), lambda i: (i, 0)),
        out_shape=jax.ShapeDtypeStruct((_NN, _D), jnp.float32),
    )(s2, c2, h, we2, be2, wh1a, wh1b, bh1, wh2, bh2)


def kernel(h, edge_index, edge_attr, We1, be1, We2, be2, Wh1, bh1, Wh2, bh2):
    src = edge_index[0].astype(jnp.int32)
    dst = edge_index[1].astype(jnp.int32)
    A, B = _proj(h, We1[0:_D], We1[_D:2 * _D])
    E = _ebias(edge_attr, We1[2 * _D:], be1.reshape(1, _D))
    S2, C2 = _sc_agg(A, B, E, src, dst)
    return _node(S2, C2.reshape(_NN, 1), h, We2, be2.reshape(1, _D),
                 Wh1[0:_D], Wh1[_D:], bh1.reshape(1, _D),
                 Wh2, bh2.reshape(1, _D))
